# baseline (speedup 1.0000x reference)
import functools

import jax
import jax.numpy as jnp
from jax import lax
from jax.experimental import pallas as pl
from jax.experimental.pallas import tpu as pltpu
from jax.experimental.pallas import tpu_sc as plsc

N = 10000
E = 320000
G = 64
H = 128

NC = 2
NS = 16
NW = NC * NS
EPAD = 327680
CHUNKS_W = 80
CH_BIG = 160
CH_SMALL = 0
NWAVE = 4
NPAD = 10112
ROWS_T = NPAD // NS
BN = 1000
GRID = N // BN

_mesh = plsc.VectorSubcoreMesh(core_axis_name="c", subcore_axis_name="s")


@functools.partial(
    pl.kernel,
    out_type=jax.ShapeDtypeStruct((NC, NPAD, H), jnp.float32),
    mesh=_mesh,
    scratch_types=[
        pltpu.VMEM((CH_BIG // NWAVE, 128), jnp.int32),
        pltpu.VMEM((CH_BIG // NWAVE, 128), jnp.int32),
        pltpu.VMEM((128, H), jnp.float32),
        pltpu.VMEM((128, H), jnp.float32),
        pltpu.VMEM_SHARED((NPAD, H), jnp.float32),
        pltpu.SemaphoreType.DMA,
        pltpu.SemaphoreType.DMA,
    ],
)
def _msg(y_hbm, row_hbm, col_hbm, zeros_hbm, out_hbm, ridx, cidx,
         rows0, rows1, acc, sem0, sem1):
    c = lax.axis_index("c")
    s = lax.axis_index("s")
    pltpu.sync_copy(zeros_hbm, acc.at[pl.ds(s * ROWS_T, ROWS_T)])

    def ring(nchunks):
        pltpu.async_copy(y_hbm.at[ridx.at[0]], rows0, sem0)

        def body2(j, carry):
            k0 = 2 * j
            k1 = k0 + 1
            pltpu.async_copy(y_hbm.at[ridx.at[k1]], rows1, sem1)
            pltpu.make_async_copy(y_hbm.at[ridx.at[k0]], rows0, sem0).wait()
            pltpu.sync_copy(rows0, acc.at[cidx.at[k0]], add=True)

            @pl.when(k1 + 1 < nchunks)
            def _pref():
                pltpu.async_copy(y_hbm.at[ridx.at[k1 + 1]], rows0, sem0)

            pltpu.make_async_copy(y_hbm.at[ridx.at[k1]], rows1, sem1).wait()
            pltpu.sync_copy(rows1, acc.at[cidx.at[k1]], add=True)
            return carry

        lax.fori_loop(0, nchunks // 2, body2, 0)

    HB = CH_BIG // NWAVE

    if CH_SMALL:
        @pl.when(c == 1)
        def _stage1():
            base = NS * CH_BIG + s * CH_SMALL
            pltpu.sync_copy(row_hbm.at[pl.ds(base, CH_SMALL)],
                            ridx.at[pl.ds(0, CH_SMALL)])
            pltpu.sync_copy(col_hbm.at[pl.ds(base, CH_SMALL)],
                            cidx.at[pl.ds(0, CH_SMALL)])
    plsc.subcore_barrier()

    @pl.when(c == 0)
    def _run0():
        for w in range(NWAVE):
            pltpu.sync_copy(row_hbm.at[pl.ds(s * CH_BIG + w * HB, HB)], ridx)
            pltpu.sync_copy(col_hbm.at[pl.ds(s * CH_BIG + w * HB, HB)], cidx)
            ring(HB)

    if CH_SMALL:
        @pl.when(c == 1)
        def _run1():
            ring(CH_SMALL)

    plsc.subcore_barrier()
    pltpu.sync_copy(acc.at[pl.ds(s * ROWS_T, ROWS_T)],
                    out_hbm.at[c, pl.ds(s * ROWS_T, ROWS_T)])


@functools.partial(
    pl.kernel,
    out_type=jax.ShapeDtypeStruct((NC, NPAD, H), jnp.float32),
    mesh=_mesh,
    scratch_types=[
        pltpu.VMEM((CHUNKS_W, 128), jnp.int32),
        pltpu.VMEM((128, H), jnp.float32),
        pltpu.VMEM_SHARED((NPAD, H), jnp.float32),
    ],
)
def _deg(col_hbm, zeros_hbm, ones_hbm, out_hbm, cidx, ones, acc):
    c = lax.axis_index("c")
    s = lax.axis_index("s")
    wid = c * NS + s

    pltpu.sync_copy(zeros_hbm, acc.at[pl.ds(s * ROWS_T, ROWS_T)])
    pltpu.sync_copy(ones_hbm, ones)
    pltpu.sync_copy(col_hbm.at[pl.ds(wid * CHUNKS_W, CHUNKS_W)], cidx)
    plsc.subcore_barrier()

    def body(j, carry):
        pltpu.sync_copy(ones, acc.at[cidx.at[j]], add=True)
        return carry

    lax.fori_loop(0, CHUNKS_W, body, 0)
    plsc.subcore_barrier()
    pltpu.sync_copy(acc.at[pl.ds(s * ROWS_T, ROWS_T)],
                    out_hbm.at[c, pl.ds(s * ROWS_T, ROWS_T)])


def _init_body(degp_ref, x_ref, winv_ref, bin_ref, wc0_ref,
               dis_ref, y0_ref):
    deg = degp_ref[0, :, 0:1] + degp_ref[1, :, 0:1] + 1.0
    dis = lax.rsqrt(deg)
    h0 = jnp.maximum(x_ref[...] * winv_ref[...] + bin_ref[...], 0.0)
    y0 = lax.dot_general(h0, wc0_ref[...], (((1,), (1,)), ((), ())),
                         preferred_element_type=jnp.float32) * dis
    dis_ref[...] = dis
    y0_ref[...] = y0


_init_call = pl.pallas_call(
    _init_body,
    grid=(GRID,),
    in_specs=[
        pl.BlockSpec((NC, BN, H), lambda i: (0, i, 0)),
        pl.BlockSpec((BN, 1), lambda i: (i, 0)),
        pl.BlockSpec((1, H), lambda i: (0, 0)),
        pl.BlockSpec((1, H), lambda i: (0, 0)),
        pl.BlockSpec((H, H), lambda i: (0, 0)),
    ],
    out_specs=[
        pl.BlockSpec((BN, 1), lambda i: (i, 0)),
        pl.BlockSpec((BN, H), lambda i: (i, 0)),
    ],
    out_shape=[
        jax.ShapeDtypeStruct((N, 1), jnp.float32),
        jax.ShapeDtypeStruct((NPAD, H), jnp.float32),
    ],
)


def _layer_body(accp_ref, y_ref, dis_ref, b_ref, w_ref, yout_ref):
    a = accp_ref[0] + accp_ref[1] + y_ref[...]
    dis = dis_ref[...]
    h = jnp.maximum(a * dis + b_ref[...], 0.0)
    yout_ref[...] = lax.dot_general(h, w_ref[...], (((1,), (1,)), ((), ())),
                                    preferred_element_type=jnp.float32) * dis


_layer_call = pl.pallas_call(
    _layer_body,
    grid=(GRID,),
    in_specs=[
        pl.BlockSpec((NC, BN, H), lambda i: (0, i, 0)),
        pl.BlockSpec((BN, H), lambda i: (i, 0)),
        pl.BlockSpec((BN, 1), lambda i: (i, 0)),
        pl.BlockSpec((1, H), lambda i: (0, 0)),
        pl.BlockSpec((H, H), lambda i: (0, 0)),
    ],
    out_specs=pl.BlockSpec((BN, H), lambda i: (i, 0)),
    out_shape=jax.ShapeDtypeStruct((NPAD, H), jnp.float32),
)


def _final_body(accp_ref, y_ref, dis_ref, b_ref, batch_ref, wout_ref, bout_ref,
                g_ref, gsum, cnt):
    i = pl.program_id(0)

    @pl.when(i == 0)
    def _zero():
        gsum[...] = jnp.zeros_like(gsum)
        cnt[...] = jnp.zeros_like(cnt)

    a = accp_ref[0] + accp_ref[1] + y_ref[...]
    h = jnp.maximum(a * dis_ref[...] + b_ref[...], 0.0)
    bb = batch_ref[0]
    onehot = (lax.broadcasted_iota(jnp.int32, (G, BN), 0) == bb
              ).astype(jnp.float32)
    gsum[...] += lax.dot_general(onehot, h, (((1,), (0,)), ((), ())),
                                 preferred_element_type=jnp.float32)
    cnt[...] += jnp.sum(onehot, axis=1, keepdims=True)

    @pl.when(i == GRID - 1)
    def _emit():
        gm = gsum[...] / jnp.maximum(cnt[...], 1.0)
        g_ref[...] = jnp.maximum(
            lax.dot_general(gm, wout_ref[...], (((1,), (1,)), ((), ())),
                            preferred_element_type=jnp.float32) + bout_ref[...],
            0.0)


_final_call = pl.pallas_call(
    _final_body,
    grid=(GRID,),
    in_specs=[
        pl.BlockSpec((NC, BN, H), lambda i: (0, i, 0)),
        pl.BlockSpec((BN, H), lambda i: (i, 0)),
        pl.BlockSpec((BN, 1), lambda i: (i, 0)),
        pl.BlockSpec((1, H), lambda i: (0, 0)),
        pl.BlockSpec((1, 1, BN), lambda i: (i, 0, 0)),
        pl.BlockSpec((H, H), lambda i: (0, 0)),
        pl.BlockSpec((1, H), lambda i: (0, 0)),
    ],
    out_specs=pl.BlockSpec((G, H), lambda i: (0, 0)),
    out_shape=jax.ShapeDtypeStruct((G, H), jnp.float32),
    scratch_shapes=[
        pltpu.VMEM((G, H), jnp.float32),
        pltpu.VMEM((G, 1), jnp.float32),
    ],
)


def kernel(x, edge_index, batch, w_in, b_in, wc0, bc0, wc1, bc1, wc2, bc2,
           w_out, b_out):
    f32 = jnp.float32
    pad = EPAD - E
    row2d = jnp.concatenate(
        [edge_index[0], jnp.zeros((pad,), edge_index.dtype)]).reshape(-1, 128)
    col2d = jnp.concatenate(
        [edge_index[1], jnp.full((pad,), N, edge_index.dtype)]).reshape(-1, 128)
    zerosH = jnp.zeros((ROWS_T, H), f32)
    ones128 = jnp.ones((128, H), f32)

    degp = _deg(col2d, zerosH, ones128)
    dis, y = _init_call(degp, x, w_in.reshape(1, H), b_in.reshape(1, H),
                        wc0)

    accp = _msg(y, row2d, col2d, zerosH)
    y = _layer_call(accp, y, dis, bc0.reshape(1, H), wc1)
    accp = _msg(y, row2d, col2d, zerosH)
    y = _layer_call(accp, y, dis, bc1.reshape(1, H), wc2)
    accp = _msg(y, row2d, col2d, zerosH)

    g = _final_call(accp, y, dis, bc2.reshape(1, H),
                    batch.reshape(GRID, 1, BN), w_out, b_out.reshape(1, H))
    return g


# speedup vs baseline: 1.3425x; 1.3425x over previous
import functools

import jax
import jax.numpy as jnp
from jax import lax
from jax.experimental import pallas as pl
from jax.experimental.pallas import tpu as pltpu
from jax.experimental.pallas import tpu_sc as plsc

N = 10000
E = 320000
G = 64
H = 128

NC = 2
NS = 16
NW = NC * NS
EPAD = 327680
CHUNKS_W = 80
CH_BIG = 144
CH_SMALL = 16
NWAVE = 3
NPAD = 10112
ROWS_T = NPAD // NS
BN = 1000
GRID = N // BN

_mesh = plsc.VectorSubcoreMesh(core_axis_name="c", subcore_axis_name="s")


@functools.partial(
    pl.kernel,
    out_type=jax.ShapeDtypeStruct((NC, NPAD, H), jnp.float32),
    mesh=_mesh,
    scratch_types=[
        pltpu.VMEM((CH_BIG // NWAVE, 128), jnp.int32),
        pltpu.VMEM((CH_BIG // NWAVE, 128), jnp.int32),
        pltpu.VMEM((128, H), jnp.float32),
        pltpu.VMEM((128, H), jnp.float32),
        pltpu.VMEM_SHARED((NPAD, H), jnp.float32),
        pltpu.SemaphoreType.DMA,
        pltpu.SemaphoreType.DMA,
    ],
)
def _msg(y_hbm, row_hbm, col_hbm, zeros_hbm, out_hbm, ridx, cidx,
         rows0, rows1, acc, sem0, sem1):
    c = lax.axis_index("c")
    s = lax.axis_index("s")
    pltpu.sync_copy(zeros_hbm, acc.at[pl.ds(s * ROWS_T, ROWS_T)])

    def ring(nchunks):
        pltpu.async_copy(y_hbm.at[ridx.at[0]], rows0, sem0)

        def body2(j, carry):
            k0 = 2 * j
            k1 = k0 + 1
            pltpu.async_copy(y_hbm.at[ridx.at[k1]], rows1, sem1)
            pltpu.make_async_copy(y_hbm.at[ridx.at[k0]], rows0, sem0).wait()
            pltpu.sync_copy(rows0, acc.at[cidx.at[k0]], add=True)

            @pl.when(k1 + 1 < nchunks)
            def _pref():
                pltpu.async_copy(y_hbm.at[ridx.at[k1 + 1]], rows0, sem0)

            pltpu.make_async_copy(y_hbm.at[ridx.at[k1]], rows1, sem1).wait()
            pltpu.sync_copy(rows1, acc.at[cidx.at[k1]], add=True)
            return carry

        lax.fori_loop(0, nchunks // 2, body2, 0)

    HB = CH_BIG // NWAVE

    if CH_SMALL:
        @pl.when(c == 1)
        def _stage1():
            base = NS * CH_BIG + s * CH_SMALL
            pltpu.sync_copy(row_hbm.at[pl.ds(base, CH_SMALL)],
                            ridx.at[pl.ds(0, CH_SMALL)])
            pltpu.sync_copy(col_hbm.at[pl.ds(base, CH_SMALL)],
                            cidx.at[pl.ds(0, CH_SMALL)])
    plsc.subcore_barrier()

    @pl.when(c == 0)
    def _run0():
        for w in range(NWAVE):
            pltpu.sync_copy(row_hbm.at[pl.ds(s * CH_BIG + w * HB, HB)], ridx)
            pltpu.sync_copy(col_hbm.at[pl.ds(s * CH_BIG + w * HB, HB)], cidx)
            ring(HB)

    if CH_SMALL:
        @pl.when(c == 1)
        def _run1():
            ring(CH_SMALL)

    plsc.subcore_barrier()
    pltpu.sync_copy(acc.at[pl.ds(s * ROWS_T, ROWS_T)],
                    out_hbm.at[c, pl.ds(s * ROWS_T, ROWS_T)])


@functools.partial(
    pl.kernel,
    out_type=jax.ShapeDtypeStruct((NC, NPAD, H), jnp.float32),
    mesh=_mesh,
    scratch_types=[
        pltpu.VMEM((CHUNKS_W, 128), jnp.int32),
        pltpu.VMEM((128, H), jnp.float32),
        pltpu.VMEM_SHARED((NPAD, H), jnp.float32),
    ],
)
def _deg(col_hbm, zeros_hbm, ones_hbm, out_hbm, cidx, ones, acc):
    c = lax.axis_index("c")
    s = lax.axis_index("s")
    wid = c * NS + s

    pltpu.sync_copy(zeros_hbm, acc.at[pl.ds(s * ROWS_T, ROWS_T)])
    pltpu.sync_copy(ones_hbm, ones)
    pltpu.sync_copy(col_hbm.at[pl.ds(wid * CHUNKS_W, CHUNKS_W)], cidx)
    plsc.subcore_barrier()

    def body(j, carry):
        pltpu.sync_copy(ones, acc.at[cidx.at[j]], add=True)
        return carry

    lax.fori_loop(0, CHUNKS_W, body, 0)
    plsc.subcore_barrier()
    pltpu.sync_copy(acc.at[pl.ds(s * ROWS_T, ROWS_T)],
                    out_hbm.at[c, pl.ds(s * ROWS_T, ROWS_T)])


def _init_body(degp_ref, x_ref, winv_ref, bin_ref, wc0_ref,
               dis_ref, y0_ref):
    deg = degp_ref[0, :, 0:1] + degp_ref[1, :, 0:1] + 1.0
    dis = lax.rsqrt(deg)
    h0 = jnp.maximum(x_ref[...] * winv_ref[...] + bin_ref[...], 0.0)
    y0 = lax.dot_general(h0, wc0_ref[...], (((1,), (1,)), ((), ())),
                         preferred_element_type=jnp.float32) * dis
    dis_ref[...] = dis
    y0_ref[...] = y0


_init_call = pl.pallas_call(
    _init_body,
    grid=(GRID,),
    in_specs=[
        pl.BlockSpec((NC, BN, H), lambda i: (0, i, 0)),
        pl.BlockSpec((BN, 1), lambda i: (i, 0)),
        pl.BlockSpec((1, H), lambda i: (0, 0)),
        pl.BlockSpec((1, H), lambda i: (0, 0)),
        pl.BlockSpec((H, H), lambda i: (0, 0)),
    ],
    out_specs=[
        pl.BlockSpec((BN, 1), lambda i: (i, 0)),
        pl.BlockSpec((BN, H), lambda i: (i, 0)),
    ],
    out_shape=[
        jax.ShapeDtypeStruct((N, 1), jnp.float32),
        jax.ShapeDtypeStruct((NPAD, H), jnp.float32),
    ],
)


def _layer_body(accp_ref, y_ref, dis_ref, b_ref, w_ref, yout_ref):
    a = accp_ref[0] + accp_ref[1] + y_ref[...]
    dis = dis_ref[...]
    h = jnp.maximum(a * dis + b_ref[...], 0.0)
    yout_ref[...] = lax.dot_general(h, w_ref[...], (((1,), (1,)), ((), ())),
                                    preferred_element_type=jnp.float32) * dis


_layer_call = pl.pallas_call(
    _layer_body,
    grid=(GRID,),
    in_specs=[
        pl.BlockSpec((NC, BN, H), lambda i: (0, i, 0)),
        pl.BlockSpec((BN, H), lambda i: (i, 0)),
        pl.BlockSpec((BN, 1), lambda i: (i, 0)),
        pl.BlockSpec((1, H), lambda i: (0, 0)),
        pl.BlockSpec((H, H), lambda i: (0, 0)),
    ],
    out_specs=pl.BlockSpec((BN, H), lambda i: (i, 0)),
    out_shape=jax.ShapeDtypeStruct((NPAD, H), jnp.float32),
)


def _final_body(accp_ref, y_ref, dis_ref, b_ref, batch_ref, wout_ref, bout_ref,
                g_ref, gsum, cnt):
    i = pl.program_id(0)

    @pl.when(i == 0)
    def _zero():
        gsum[...] = jnp.zeros_like(gsum)
        cnt[...] = jnp.zeros_like(cnt)

    a = accp_ref[0] + accp_ref[1] + y_ref[...]
    h = jnp.maximum(a * dis_ref[...] + b_ref[...], 0.0)
    bb = batch_ref[0]
    onehot = (lax.broadcasted_iota(jnp.int32, (G, BN), 0) == bb
              ).astype(jnp.float32)
    gsum[...] += lax.dot_general(onehot, h, (((1,), (0,)), ((), ())),
                                 preferred_element_type=jnp.float32)
    cnt[...] += jnp.sum(onehot, axis=1, keepdims=True)

    @pl.when(i == GRID - 1)
    def _emit():
        gm = gsum[...] / jnp.maximum(cnt[...], 1.0)
        g_ref[...] = jnp.maximum(
            lax.dot_general(gm, wout_ref[...], (((1,), (1,)), ((), ())),
                            preferred_element_type=jnp.float32) + bout_ref[...],
            0.0)


_final_call = pl.pallas_call(
    _final_body,
    grid=(GRID,),
    in_specs=[
        pl.BlockSpec((NC, BN, H), lambda i: (0, i, 0)),
        pl.BlockSpec((BN, H), lambda i: (i, 0)),
        pl.BlockSpec((BN, 1), lambda i: (i, 0)),
        pl.BlockSpec((1, H), lambda i: (0, 0)),
        pl.BlockSpec((1, 1, BN), lambda i: (i, 0, 0)),
        pl.BlockSpec((H, H), lambda i: (0, 0)),
        pl.BlockSpec((1, H), lambda i: (0, 0)),
    ],
    out_specs=pl.BlockSpec((G, H), lambda i: (0, 0)),
    out_shape=jax.ShapeDtypeStruct((G, H), jnp.float32),
    scratch_shapes=[
        pltpu.VMEM((G, H), jnp.float32),
        pltpu.VMEM((G, 1), jnp.float32),
    ],
)


def kernel(x, edge_index, batch, w_in, b_in, wc0, bc0, wc1, bc1, wc2, bc2,
           w_out, b_out):
    f32 = jnp.float32
    pad = EPAD - E
    row2d = jnp.concatenate(
        [edge_index[0], jnp.zeros((pad,), edge_index.dtype)]).reshape(-1, 128)
    col2d = jnp.concatenate(
        [edge_index[1], jnp.full((pad,), N, edge_index.dtype)]).reshape(-1, 128)
    zerosH = jnp.zeros((ROWS_T, H), f32)
    ones128 = jnp.ones((128, H), f32)

    degp = _deg(col2d, zerosH, ones128)
    dis, y = _init_call(degp, x, w_in.reshape(1, H), b_in.reshape(1, H),
                        wc0)

    accp = _msg(y, row2d, col2d, zerosH)
    y = _layer_call(accp, y, dis, bc0.reshape(1, H), wc1)
    accp = _msg(y, row2d, col2d, zerosH)
    y = _layer_call(accp, y, dis, bc1.reshape(1, H), wc2)
    accp = _msg(y, row2d, col2d, zerosH)

    g = _final_call(accp, y, dis, bc2.reshape(1, H),
                    batch.reshape(GRID, 1, BN), w_out, b_out.reshape(1, H))
    return g
